# baseline (speedup 1.0000x reference)
import functools

import jax
import jax.numpy as jnp
from jax import lax
from jax.experimental import pallas as pl
from jax.experimental.pallas import tpu as pltpu
from jax.experimental.pallas import tpu_sc as plsc

B = 16384
POSE = 8
VOCAB = 1500
EMB = 64
EMBP = 128
EPS = 1e-5

_NC, _NS, _L = 2, 16, 16
_NW = _NC * _NS
_BPW = B // _NW
_CH = 128
_NCH = _BPW // _CH

_BS = 4096
_NB = B // _BS


def _emb_gather(emb_table, idx2d):
    mesh = plsc.VectorSubcoreMesh(core_axis_name="c", subcore_axis_name="s")

    @functools.partial(
        pl.kernel,
        mesh=mesh,
        out_type=jax.ShapeDtypeStruct((B, EMBP), jnp.float32),
        scratch_types=[
            pltpu.VMEM((_NCH, _CH), jnp.int32),
            pltpu.VMEM((_BPW, EMBP), jnp.float32),
            pltpu.SemaphoreType.DMA,
        ],
    )
    def k(table_hbm, idx_hbm, out_hbm, idx_v, rows_v, sem):
        wid = lax.axis_index("s") * _NC + lax.axis_index("c")
        pltpu.sync_copy(idx_hbm.at[pl.ds(wid * _NCH, _NCH)], idx_v)
        handles = []
        for j in range(_NCH):
            handles.append(
                pltpu.async_copy(
                    table_hbm.at[idx_v.at[j]],
                    rows_v.at[pl.ds(j * _CH, _CH)],
                    sem,
                )
            )
        for h in handles:
            h.wait()
        pltpu.sync_copy(rows_v, out_hbm.at[pl.ds(wid * _BPW, _BPW)])

    return k(emb_table, idx2d)


def _dotb(a, b):
    return jnp.dot(a.astype(jnp.bfloat16), b.astype(jnp.bfloat16),
                   preferred_element_type=jnp.float32)


def _mlp_body(pose_ref, emb_ref, W1_ref, b1_ref, g1_ref, be1_ref,
              W2_ref, b2_ref, W3a_ref, W3b_ref, b3_ref, g2_ref, be2_ref,
              W4_ref, b4_ref, W5_ref, b5_ref, W6_ref, b6_ref,
              out_ref, x1_s, h3_s, s1_s, q1_s, s2_s, q2_s):
    p = pl.program_id(0)
    i = pl.program_id(1)
    row = i * _BS

    @pl.when(p == 0)
    def _phase0():
        @pl.when(i == 0)
        def _init():
            s1_s[...] = jnp.zeros_like(s1_s)
            q1_s[...] = jnp.zeros_like(q1_s)

        x1 = lax.dot_general(pose_ref[...], W1_ref[...],
                             (((0,), (0,)), ((), ())),
                             preferred_element_type=jnp.float32) + b1_ref[...]
        x1 = jnp.maximum(x1, 0.0)
        x1_s[pl.ds(row, _BS), :] = x1
        s1_s[...] += jnp.sum(x1, axis=0, keepdims=True)
        q1_s[...] += jnp.sum(x1 * x1, axis=0, keepdims=True)

    @pl.when(p == 1)
    def _phase1():
        @pl.when(i == 0)
        def _init():
            s2_s[...] = jnp.zeros_like(s2_s)
            q2_s[...] = jnp.zeros_like(q2_s)

        m1 = s1_s[...] * (1.0 / B)
        v1 = q1_s[...] * (1.0 / B) - m1 * m1
        sc1 = g1_ref[...] * lax.rsqrt(v1 + EPS)
        sh1 = be1_ref[...] - m1 * sc1
        x1 = x1_s[pl.ds(row, _BS), :] * sc1 + sh1
        x2 = _dotb(x1, W2_ref[...]) + b2_ref[...]
        x2 = jnp.maximum(x2, 0.0)
        h3 = (_dotb(x2, W3a_ref[...])
              + _dotb(emb_ref[...], W3b_ref[...])
              + b3_ref[...])
        h3 = jnp.maximum(h3, 0.0)
        h3_s[pl.ds(row, _BS), :] = h3
        s2_s[...] += jnp.sum(h3, axis=0, keepdims=True)
        q2_s[...] += jnp.sum(h3 * h3, axis=0, keepdims=True)

    @pl.when(p == 2)
    def _phase2():
        m2 = s2_s[...] * (1.0 / B)
        v2 = q2_s[...] * (1.0 / B) - m2 * m2
        sc2 = g2_ref[...] * lax.rsqrt(v2 + EPS)
        sh2 = be2_ref[...] - m2 * sc2
        h3 = h3_s[pl.ds(row, _BS), :] * sc2 + sh2
        h4 = _dotb(h3, W4_ref[...]) + b4_ref[...]
        h4 = jnp.maximum(h4, 0.0)
        h5 = _dotb(h4, W5_ref[...]) + b5_ref[...]
        h5 = jnp.maximum(h5, 0.0)
        z = lax.dot_general(W6_ref[...], h5, (((0,), (1,)), ((), ())),
                            preferred_element_type=jnp.float32) + b6_ref[...]
        out_ref[...] = jax.nn.sigmoid(z)


def _mlp(pose, emb, W1, b1, g1, be1, W2, b2, W3a, W3b, b3, g2, be2,
         W4, b4, W5, b5, W6, b6, interpret=False):
    full = lambda shape: pl.BlockSpec(shape, lambda p, i: (0, 0))
    return pl.pallas_call(
        _mlp_body,
        grid=(3, _NB),
        in_specs=[
            pl.BlockSpec((POSE, _BS), lambda p, i: (0, jnp.where(p == 0, i, 0))),
            pl.BlockSpec((_BS, EMBP), lambda p, i: (jnp.where(p == 1, i, 0), 0)),
            full((POSE, 128)), full((1, 128)), full((1, 128)), full((1, 128)),
            full((128, 64)), full((1, 64)),
            full((64, 128)), full((EMBP, 128)), full((1, 128)),
            full((1, 128)), full((1, 128)),
            full((128, 64)), full((1, 64)),
            full((64, 32)), full((1, 32)),
            full((32, 1)), full((1, 1)),
        ],
        out_specs=pl.BlockSpec((1, _BS), lambda p, i: (0, jnp.where(p == 2, i, 0))),
        out_shape=jax.ShapeDtypeStruct((1, B), jnp.float32),
        scratch_shapes=[
            pltpu.VMEM((B, 128), jnp.float32),
            pltpu.VMEM((B, 128), jnp.float32),
            pltpu.VMEM((1, 128), jnp.float32),
            pltpu.VMEM((1, 128), jnp.float32),
            pltpu.VMEM((1, 128), jnp.float32),
            pltpu.VMEM((1, 128), jnp.float32),
        ],
        interpret=interpret,
    )(pose, emb, W1, b1, g1, be1, W2, b2, W3a, W3b, b3, g2, be2,
      W4, b4, W5, b5, W6, b6)


def kernel(pose_features, exercise_id, emb_table, W1, b1, g1, be1, W2, b2,
           W3, b3, g2, be2, W4, b4, W5, b5, W6, b6):
    idx2d = exercise_id.astype(jnp.int32).reshape(_NW * _NCH, _CH)
    table_p = jnp.pad(emb_table, ((0, 0), (0, EMBP - EMB)))
    emb = _emb_gather(table_p, idx2d)
    r = lambda a: a.reshape(1, -1)
    W3b_p = jnp.pad(W3[64:], ((0, EMBP - EMB), (0, 0)))
    out_t = _mlp(pose_features.T, emb,
                 W1, r(b1), r(g1), r(be1),
                 W2, r(b2),
                 W3[:64], W3b_p, r(b3),
                 r(g2), r(be2),
                 W4, r(b4), W5, r(b5), W6, r(b6))
    return out_t.reshape(B, 1)


# speedup vs baseline: 1.0429x; 1.0429x over previous
import functools

import jax
import jax.numpy as jnp
from jax import lax
from jax.experimental import pallas as pl
from jax.experimental.pallas import tpu as pltpu
from jax.experimental.pallas import tpu_sc as plsc

B = 16384
POSE = 8
VOCAB = 1500
EMB = 64
EMBP = 128
EPS = 1e-5

_NC, _NS, _L = 2, 16, 16
_NW = _NC * _NS
_BPW = B // _NW
_CH = 128
_NCH = _BPW // _CH

_BS = 8192
_NB = B // _BS


def _emb_gather(emb_table, idx2d):
    mesh = plsc.VectorSubcoreMesh(core_axis_name="c", subcore_axis_name="s")

    @functools.partial(
        pl.kernel,
        mesh=mesh,
        out_type=jax.ShapeDtypeStruct((B, EMBP), jnp.float32),
        scratch_types=[
            pltpu.VMEM((_NCH, _CH), jnp.int32),
            pltpu.VMEM((_BPW, EMBP), jnp.float32),
            pltpu.SemaphoreType.DMA,
        ],
    )
    def k(table_hbm, idx_hbm, out_hbm, idx_v, rows_v, sem):
        wid = lax.axis_index("s") * _NC + lax.axis_index("c")
        pltpu.sync_copy(idx_hbm.at[pl.ds(wid * _NCH, _NCH)], idx_v)
        handles = []
        for j in range(_NCH):
            handles.append(
                pltpu.async_copy(
                    table_hbm.at[idx_v.at[j]],
                    rows_v.at[pl.ds(j * _CH, _CH)],
                    sem,
                )
            )
        for h in handles:
            h.wait()
        pltpu.sync_copy(rows_v, out_hbm.at[pl.ds(wid * _BPW, _BPW)])

    return k(emb_table, idx2d)


def _dotb(a, b):
    return jnp.dot(a.astype(jnp.bfloat16), b.astype(jnp.bfloat16),
                   preferred_element_type=jnp.float32)


def _mlp_body(pose_ref, emb_ref, W1_ref, b1_ref, g1_ref, be1_ref,
              W2_ref, b2_ref, W3a_ref, W3b_ref, b3_ref, g2_ref, be2_ref,
              W4_ref, b4_ref, W5_ref, b5_ref, W6_ref, b6_ref,
              out_ref, x1_s, h3_s, s1_s, q1_s, s2_s, q2_s):
    p = pl.program_id(0)
    i = pl.program_id(1)
    row = i * _BS

    @pl.when(p == 0)
    def _phase0():
        @pl.when(i == 0)
        def _init():
            s1_s[...] = jnp.zeros_like(s1_s)
            q1_s[...] = jnp.zeros_like(q1_s)

        x1 = lax.dot_general(pose_ref[...], W1_ref[...],
                             (((0,), (0,)), ((), ())),
                             preferred_element_type=jnp.float32) + b1_ref[...]
        x1 = jnp.maximum(x1, 0.0)
        x1_s[pl.ds(row, _BS), :] = x1
        s1_s[...] += jnp.sum(x1, axis=0, keepdims=True)
        q1_s[...] += jnp.sum(x1 * x1, axis=0, keepdims=True)

    @pl.when(p == 1)
    def _phase1():
        @pl.when(i == 0)
        def _init():
            s2_s[...] = jnp.zeros_like(s2_s)
            q2_s[...] = jnp.zeros_like(q2_s)

        m1 = s1_s[...] * (1.0 / B)
        v1 = q1_s[...] * (1.0 / B) - m1 * m1
        sc1 = g1_ref[...] * lax.rsqrt(v1 + EPS)
        sh1 = be1_ref[...] - m1 * sc1
        x1 = x1_s[pl.ds(row, _BS), :] * sc1 + sh1
        x2 = _dotb(x1, W2_ref[...]) + b2_ref[...]
        x2 = jnp.maximum(x2, 0.0)
        h3 = (_dotb(x2, W3a_ref[...])
              + _dotb(emb_ref[...], W3b_ref[...])
              + b3_ref[...])
        h3 = jnp.maximum(h3, 0.0)
        h3_s[pl.ds(row, _BS), :] = h3
        s2_s[...] += jnp.sum(h3, axis=0, keepdims=True)
        q2_s[...] += jnp.sum(h3 * h3, axis=0, keepdims=True)

    @pl.when(p == 2)
    def _phase2():
        m2 = s2_s[...] * (1.0 / B)
        v2 = q2_s[...] * (1.0 / B) - m2 * m2
        sc2 = g2_ref[...] * lax.rsqrt(v2 + EPS)
        sh2 = be2_ref[...] - m2 * sc2
        h3 = h3_s[pl.ds(row, _BS), :] * sc2 + sh2
        h4 = _dotb(h3, W4_ref[...]) + b4_ref[...]
        h4 = jnp.maximum(h4, 0.0)
        h5 = _dotb(h4, W5_ref[...]) + b5_ref[...]
        h5 = jnp.maximum(h5, 0.0)
        z = lax.dot_general(W6_ref[...], h5, (((0,), (1,)), ((), ())),
                            preferred_element_type=jnp.float32) + b6_ref[...]
        out_ref[...] = jax.nn.sigmoid(z)


def _mlp(pose, emb, W1, b1, g1, be1, W2, b2, W3a, W3b, b3, g2, be2,
         W4, b4, W5, b5, W6, b6, interpret=False):
    full = lambda shape: pl.BlockSpec(shape, lambda p, i: (0, 0))
    return pl.pallas_call(
        _mlp_body,
        grid=(3, _NB),
        in_specs=[
            pl.BlockSpec((POSE, _BS), lambda p, i: (0, jnp.where(p == 0, i, 0))),
            pl.BlockSpec((_BS, EMBP), lambda p, i: (jnp.where(p == 1, i, 0), 0)),
            full((POSE, 128)), full((1, 128)), full((1, 128)), full((1, 128)),
            full((128, 64)), full((1, 64)),
            full((64, 128)), full((EMBP, 128)), full((1, 128)),
            full((1, 128)), full((1, 128)),
            full((128, 64)), full((1, 64)),
            full((64, 32)), full((1, 32)),
            full((32, 1)), full((1, 1)),
        ],
        out_specs=pl.BlockSpec((1, _BS), lambda p, i: (0, jnp.where(p == 2, i, 0))),
        out_shape=jax.ShapeDtypeStruct((1, B), jnp.float32),
        scratch_shapes=[
            pltpu.VMEM((B, 128), jnp.float32),
            pltpu.VMEM((B, 128), jnp.float32),
            pltpu.VMEM((1, 128), jnp.float32),
            pltpu.VMEM((1, 128), jnp.float32),
            pltpu.VMEM((1, 128), jnp.float32),
            pltpu.VMEM((1, 128), jnp.float32),
        ],
        interpret=interpret,
    )(pose, emb, W1, b1, g1, be1, W2, b2, W3a, W3b, b3, g2, be2,
      W4, b4, W5, b5, W6, b6)


def kernel(pose_features, exercise_id, emb_table, W1, b1, g1, be1, W2, b2,
           W3, b3, g2, be2, W4, b4, W5, b5, W6, b6):
    idx2d = exercise_id.astype(jnp.int32).reshape(_NW * _NCH, _CH)
    table_p = jnp.pad(emb_table, ((0, 0), (0, EMBP - EMB)))
    emb = _emb_gather(table_p, idx2d)
    r = lambda a: a.reshape(1, -1)
    W3b_p = jnp.pad(W3[64:], ((0, EMBP - EMB), (0, 0)))
    out_t = _mlp(pose_features.T, emb,
                 W1, r(b1), r(g1), r(be1),
                 W2, r(b2),
                 W3[:64], W3b_p, r(b3),
                 r(g2), r(be2),
                 W4, r(b4), W5, r(b5), W6, r(b6))
    return out_t.reshape(B, 1)


# speedup vs baseline: 1.1049x; 1.0595x over previous
import functools

import jax
import jax.numpy as jnp
from jax import lax
from jax.experimental import pallas as pl
from jax.experimental.pallas import tpu as pltpu
from jax.experimental.pallas import tpu_sc as plsc

B = 16384
POSE = 8
VOCAB = 1500
EMB = 64
EMBP = 128
EPS = 1e-5

_NC, _NS, _L = 2, 16, 16
_NW = _NC * _NS
_BPW = B // _NW
_CH = 128
_NCH = _BPW // _CH

_BS = 8192
_NB = B // _BS


def _emb_gather(emb_table, idx2d):
    mesh = plsc.VectorSubcoreMesh(core_axis_name="c", subcore_axis_name="s")

    @functools.partial(
        pl.kernel,
        mesh=mesh,
        out_type=jax.ShapeDtypeStruct((B, EMBP), jnp.float32),
        scratch_types=[
            pltpu.VMEM((_NCH, _CH), jnp.int32),
            pltpu.VMEM((_BPW, EMBP), jnp.float32),
            pltpu.SemaphoreType.DMA,
        ],
    )
    def k(table_hbm, idx_hbm, out_hbm, idx_v, rows_v, sem):
        wid = lax.axis_index("s") * _NC + lax.axis_index("c")
        pltpu.sync_copy(idx_hbm.at[pl.ds(wid * _NCH, _NCH)], idx_v)
        handles = []
        for j in range(_NCH):
            handles.append(
                pltpu.async_copy(
                    table_hbm.at[idx_v.at[j]],
                    rows_v.at[pl.ds(j * _CH, _CH)],
                    sem,
                )
            )
        for h in handles:
            h.wait()
        pltpu.sync_copy(rows_v, out_hbm.at[pl.ds(wid * _BPW, _BPW)])

    return k(emb_table, idx2d)


def _dotb(a, b):
    return jnp.dot(a.astype(jnp.bfloat16), b.astype(jnp.bfloat16),
                   preferred_element_type=jnp.float32)


def _x1(pose_t, W1_ref, b1_ref):
    x1 = lax.dot_general(pose_t, W1_ref[...],
                         (((0,), (0,)), ((), ())),
                         preferred_element_type=jnp.float32) + b1_ref[...]
    return jnp.maximum(x1, 0.0)


def _stats_body(pose_ref, W1_ref, b1_ref, out_ref, s_s, q_s):
    i = pl.program_id(0)

    @pl.when(i == 0)
    def _init():
        s_s[...] = jnp.zeros_like(s_s)
        q_s[...] = jnp.zeros_like(q_s)

    x1 = _x1(pose_ref[...], W1_ref, b1_ref)
    s_s[...] += jnp.sum(x1, axis=0, keepdims=True)
    q_s[...] += jnp.sum(x1 * x1, axis=0, keepdims=True)

    @pl.when(i == _NB - 1)
    def _fin():
        out_ref[0:1, :] = s_s[...]
        out_ref[1:2, :] = q_s[...]


def _stats(pose_t, W1, b1, interpret=False):
    full = lambda shape: pl.BlockSpec(shape, lambda i: (0, 0))
    return pl.pallas_call(
        _stats_body,
        grid=(_NB,),
        in_specs=[
            pl.BlockSpec((POSE, _BS), lambda i: (0, i)),
            full((POSE, 128)), full((1, 128)),
        ],
        out_specs=full((2, 128)),
        out_shape=jax.ShapeDtypeStruct((2, 128), jnp.float32),
        scratch_shapes=[
            pltpu.VMEM((1, 128), jnp.float32),
            pltpu.VMEM((1, 128), jnp.float32),
        ],
        interpret=interpret,
    )(pose_t, W1, b1)


def _mlp_body(pose_ref, emb_ref, st1_ref, W1_ref, b1_ref, g1_ref, be1_ref,
              W2_ref, b2_ref, W3a_ref, W3b_ref, b3_ref, g2_ref, be2_ref,
              W4_ref, b4_ref, W5_ref, b5_ref, W6_ref, b6_ref,
              out_ref, h3_s, s2_s, q2_s):
    p = pl.program_id(0)
    i = pl.program_id(1)
    row = i * _BS

    @pl.when(p == 0)
    def _phase0():
        @pl.when(i == 0)
        def _init():
            s2_s[...] = jnp.zeros_like(s2_s)
            q2_s[...] = jnp.zeros_like(q2_s)

        m1 = st1_ref[0:1, :] * (1.0 / B)
        v1 = st1_ref[1:2, :] * (1.0 / B) - m1 * m1
        sc1 = g1_ref[...] * lax.rsqrt(v1 + EPS)
        sh1 = be1_ref[...] - m1 * sc1
        x1 = _x1(pose_ref[...], W1_ref, b1_ref) * sc1 + sh1
        x2 = _dotb(x1, W2_ref[...]) + b2_ref[...]
        x2 = jnp.maximum(x2, 0.0)
        h3 = (_dotb(x2, W3a_ref[...])
              + _dotb(emb_ref[...], W3b_ref[...])
              + b3_ref[...])
        h3 = jnp.maximum(h3, 0.0)
        h3_s[pl.ds(row, _BS), :] = h3
        s2_s[...] += jnp.sum(h3, axis=0, keepdims=True)
        q2_s[...] += jnp.sum(h3 * h3, axis=0, keepdims=True)

    @pl.when(p == 1)
    def _phase1():
        m2 = s2_s[...] * (1.0 / B)
        v2 = q2_s[...] * (1.0 / B) - m2 * m2
        sc2 = g2_ref[...] * lax.rsqrt(v2 + EPS)
        sh2 = be2_ref[...] - m2 * sc2
        h3 = h3_s[pl.ds(row, _BS), :] * sc2 + sh2
        h4 = _dotb(h3, W4_ref[...]) + b4_ref[...]
        h4 = jnp.maximum(h4, 0.0)
        h5 = _dotb(h4, W5_ref[...]) + b5_ref[...]
        h5 = jnp.maximum(h5, 0.0)
        z = lax.dot_general(W6_ref[...], h5, (((0,), (1,)), ((), ())),
                            preferred_element_type=jnp.float32) + b6_ref[...]
        out_ref[...] = jax.nn.sigmoid(z)


def _mlp(pose_t, emb, st1, W1, b1, g1, be1, W2, b2, W3a, W3b, b3, g2, be2,
         W4, b4, W5, b5, W6, b6, interpret=False):
    full = lambda shape: pl.BlockSpec(shape, lambda p, i: (0, 0))
    return pl.pallas_call(
        _mlp_body,
        grid=(2, _NB),
        in_specs=[
            pl.BlockSpec((POSE, _BS), lambda p, i: (0, jnp.where(p == 0, i, 0))),
            pl.BlockSpec((_BS, EMBP), lambda p, i: (jnp.where(p == 0, i, 0), 0)),
            full((2, 128)),
            full((POSE, 128)), full((1, 128)), full((1, 128)), full((1, 128)),
            full((128, 64)), full((1, 64)),
            full((64, 128)), full((EMBP, 128)), full((1, 128)),
            full((1, 128)), full((1, 128)),
            full((128, 64)), full((1, 64)),
            full((64, 32)), full((1, 32)),
            full((32, 1)), full((1, 1)),
        ],
        out_specs=pl.BlockSpec((1, _BS), lambda p, i: (0, jnp.where(p == 1, i, 0))),
        out_shape=jax.ShapeDtypeStruct((1, B), jnp.float32),
        scratch_shapes=[
            pltpu.VMEM((B, 128), jnp.float32),
            pltpu.VMEM((1, 128), jnp.float32),
            pltpu.VMEM((1, 128), jnp.float32),
        ],
        interpret=interpret,
    )(pose_t, emb, st1, W1, b1, g1, be1, W2, b2, W3a, W3b, b3, g2, be2,
      W4, b4, W5, b5, W6, b6)


def kernel(pose_features, exercise_id, emb_table, W1, b1, g1, be1, W2, b2,
           W3, b3, g2, be2, W4, b4, W5, b5, W6, b6):
    idx2d = exercise_id.astype(jnp.int32).reshape(_NW * _NCH, _CH)
    table_p = jnp.pad(emb_table, ((0, 0), (0, EMBP - EMB)))
    emb = _emb_gather(table_p, idx2d)
    r = lambda a: a.reshape(1, -1)
    W3b_p = jnp.pad(W3[64:], ((0, EMBP - EMB), (0, 0)))
    pose_t = pose_features.T
    st1 = _stats(pose_t, W1, r(b1))
    out_t = _mlp(pose_t, emb, st1,
                 W1, r(b1), r(g1), r(be1),
                 W2, r(b2),
                 W3[:64], W3b_p, r(b3),
                 r(g2), r(be2),
                 W4, r(b4), W5, r(b5), W6, r(b6))
    return out_t.reshape(B, 1)
